# Initial kernel scaffold; baseline (speedup 1.0000x reference)
import jax
import jax.numpy as jnp
from jax.experimental import pallas as pl


def kernel(x, edge_index, edge_in, in_w, edge_out, out_w, edge_index2, edge_weight, edge_weight2, lin1_w, ln_w, ln_b, conv1_w, conv1_b, conv2_w, conv2_b, conv_w, conv_b, bn_gamma, bn_beta):
    raise NotImplementedError("write your pallas kernel here")


# speedup vs baseline: 8.3105x; 8.3105x over previous
import functools

import jax
import jax.numpy as jnp
from jax import lax
from jax.experimental import pallas as pl
from jax.experimental.pallas import tpu as pltpu
from jax.experimental.pallas import tpu_sc as plsc

_N = 10000
_E = 320000
_D = 128
_BN_EPS = 1e-5

_NPAD = 10240
_NP3 = 3 * _NPAD
_NP4 = 4 * _NPAD

_NC = 2
_NS = 16

_DEG_ROWS = 7680
_DEG_RPT = 240
_DEG_CKR = 24
_DEG_NCH = 10

_PROP_ROWS = 12800
_PROP_RPT = 800
_PROP_CKR = 40
_PROP_NCH = 20
_DH = _D // _NC

_SLICE3 = _NP3 // _NS
_NOUT = _NPAD
_ROWS_OUT = _NOUT // _NS


def _deg_body(idx_hbm, w_hbm, out_hbm, idxbuf, wbuf, stage, sdeg):
    c = lax.axis_index("c")
    s = lax.axis_index("s")

    def zfill(i, _):
        stage[pl.ds(i * 16, 16)] = jnp.zeros((16,), jnp.float32)
        return 0

    lax.fori_loop(0, _SLICE3 // 16, zfill, 0)
    pltpu.sync_copy(stage, sdeg.at[pl.ds(s * _SLICE3, _SLICE3)])
    plsc.subcore_barrier()

    base = (c * _NS + s) * _DEG_RPT

    def chunk(ch, _):
        rb = base + ch * _DEG_CKR
        pltpu.sync_copy(idx_hbm.at[pl.ds(rb, _DEG_CKR)], idxbuf)
        pltpu.sync_copy(w_hbm.at[pl.ds(rb, _DEG_CKR)], wbuf)

        def row(r, _):
            pltpu.sync_copy(wbuf.at[r], sdeg.at[idxbuf.at[r]], add=True)
            return 0

        lax.fori_loop(0, _DEG_CKR, row, 0)
        return 0

    lax.fori_loop(0, _DEG_NCH, chunk, 0)
    plsc.subcore_barrier()
    pltpu.sync_copy(sdeg.at[pl.ds(s * _SLICE3, _SLICE3)],
                    out_hbm.at[c, pl.ds(s * _SLICE3, _SLICE3)])


def _prop_body(f3, d4h, rowfh, colh, grh, gch, wh, pout,
               d4buf, rfbuf, clbuf, grbuf, gcbuf, wbuf, rows_v, sout):
    c = lax.axis_index("c")
    s = lax.axis_index("s")

    def zfill(i, _):
        rows_v[i // 4, pl.ds((i % 4) * 16, 16)] = jnp.zeros((16,), jnp.float32)
        return 0

    lax.fori_loop(0, 128 * 4, zfill, 0)

    def zcopy(k, _):
        pltpu.sync_copy(rows_v,
                        sout.at[pl.ds(s * _ROWS_OUT + k * 128, 128)])
        return 0

    lax.fori_loop(0, 5, zcopy, 0)
    pltpu.sync_copy(d4h, d4buf)
    plsc.subcore_barrier()

    base = s * _PROP_RPT

    def chunk(ch, _):
        rb = base + ch * _PROP_CKR
        pltpu.sync_copy(rowfh.at[pl.ds(rb, _PROP_CKR)], rfbuf)
        pltpu.sync_copy(colh.at[pl.ds(rb, _PROP_CKR)], clbuf)
        pltpu.sync_copy(grh.at[pl.ds(rb, _PROP_CKR)], grbuf)
        pltpu.sync_copy(gch.at[pl.ds(rb, _PROP_CKR)], gcbuf)
        pltpu.sync_copy(wh.at[pl.ds(rb, _PROP_CKR)], wbuf)

        def nrm(i, _):
            r = i // 8
            g = (i % 8) * 16
            a = plsc.load_gather(d4buf, [grbuf[r, pl.ds(g, 16)]])
            b = plsc.load_gather(d4buf, [gcbuf[r, pl.ds(g, 16)]])
            wbuf[r, pl.ds(g, 16)] = a * wbuf[r, pl.ds(g, 16)] * b
            return 0

        lax.fori_loop(0, _PROP_CKR * 8, nrm, 0)

        def row(r, _):
            pltpu.sync_copy(f3.at[c].at[rfbuf.at[r]], rows_v)

            def scale(e, _):
                bc = plsc.load_gather(
                    wbuf, [lax.broadcast(r, (16,)), lax.broadcast(e, (16,))])
                for k in range(4):
                    rows_v[e, pl.ds(k * 16, 16)] = (
                        rows_v[e, pl.ds(k * 16, 16)] * bc)
                return 0

            lax.fori_loop(0, 128, scale, 0)
            pltpu.sync_copy(rows_v, sout.at[clbuf.at[r]], add=True)
            return 0

        lax.fori_loop(0, _PROP_CKR, row, 0)
        return 0

    lax.fori_loop(0, _PROP_NCH, chunk, 0)
    plsc.subcore_barrier()

    def cpout(k, _):
        pltpu.sync_copy(sout.at[pl.ds(s * _ROWS_OUT + k * 128, 128)],
                        pout.at[c, pl.ds(s * _ROWS_OUT + k * 128, 128)])
        return 0

    lax.fori_loop(0, 5, cpout, 0)


def _dis_body(deg_ref, out_ref):
    dsum = deg_ref[0:240, :] + deg_ref[240:480, :]
    dis = jnp.where(dsum > 0.0, lax.rsqrt(jnp.where(dsum > 0.0, dsum, 1.0)),
                    0.0)
    out_ref[...] = jnp.concatenate(
        [dis, jnp.ones((80, _D), jnp.float32)], axis=0)


def _fmm_body(x_ref, w_ref, out_ref):
    out_ref[0] = jnp.dot(x_ref[...], w_ref[0],
                         preferred_element_type=jnp.float32)


def _final_body(x_ref, p_ref, lnw_ref, convw_ref, bsum_ref, convb_ref,
                gam_ref, bet_ref, out_ref):
    s = gam_ref[...] * (1.0 / (1.0 + _BN_EPS) ** 0.5)
    wp = convw_ref[...] * s
    h = (jnp.dot(x_ref[...], lnw_ref[...], preferred_element_type=jnp.float32)
         + p_ref[...] + bsum_ref[...])
    out_ref[...] = (jnp.dot(h, wp, preferred_element_type=jnp.float32)
                    + convb_ref[...] * s + bet_ref[...])


def kernel(x, edge_index, edge_in, in_w, edge_out, out_w, edge_index2,
           edge_weight, edge_weight2, lin1_w, ln_w, ln_b, conv1_w, conv1_b,
           conv2_w, conv2_b, conv_w, conv_b, bn_gamma, bn_beta):
    i32 = jnp.int32
    f32 = jnp.float32

    ones_e = jnp.ones((_E,), f32)
    rows = jnp.concatenate([edge_index[0], edge_in[0], edge_out[0],
                            edge_index[0], edge_index2[0]])
    cols = jnp.concatenate([edge_index[1], edge_in[1], edge_out[1],
                            edge_index[1], edge_index2[1]])
    wall = jnp.concatenate([ones_e, in_w, out_w, edge_weight, edge_weight2])
    plane = jnp.repeat(jnp.array([0, 1, 2, 3, 3], i32), _E)
    foff = jnp.repeat(jnp.array([0, 0, 0, _N, 2 * _N], i32), _E)
    rowf = rows + foff
    gr = rows + plane * _NPAD
    gc = cols + plane * _NPAD

    npad_e = _PROP_ROWS * _D - 5 * _E
    ar = jnp.arange(npad_e, dtype=i32)
    rowf = jnp.concatenate([rowf, ar % (3 * _N)]).reshape(_PROP_ROWS, _D)
    colp = jnp.concatenate([cols, ar % _N]).reshape(_PROP_ROWS, _D)
    grp = jnp.concatenate([gr, ar % _NP4]).reshape(_PROP_ROWS, _D)
    gcp = jnp.concatenate([gc, ar % _NP4]).reshape(_PROP_ROWS, _D)
    wp_ = jnp.concatenate([wall, jnp.zeros((npad_e,), f32)]
                          ).reshape(_PROP_ROWS, _D)

    degidx = rows[:3 * _E] + plane[:3 * _E] * _NPAD
    degw = wall[:3 * _E]
    npad_d = _DEG_ROWS * _D - 3 * _E
    ard = jnp.arange(npad_d, dtype=i32)
    degidx = jnp.concatenate([degidx, ard % _NP3]).reshape(_DEG_ROWS, _D)
    degw = jnp.concatenate([degw, jnp.zeros((npad_d,), f32)]
                           ).reshape(_DEG_ROWS, _D)

    deg_part = pl.kernel(
        _deg_body,
        out_type=jax.ShapeDtypeStruct((_NC, _NP3), f32),
        mesh=plsc.VectorSubcoreMesh(core_axis_name="c", subcore_axis_name="s",
                                    num_cores=_NC, num_subcores=_NS),
        compiler_params=pltpu.CompilerParams(needs_layout_passes=False),
        scratch_types=[
            pltpu.VMEM((_DEG_CKR, _D), i32),
            pltpu.VMEM((_DEG_CKR, _D), f32),
            pltpu.VMEM((_SLICE3,), f32),
            pltpu.VMEM_SHARED((_NP3,), f32),
        ],
    )(degidx, degw)

    d4 = pl.pallas_call(
        _dis_body,
        out_shape=jax.ShapeDtypeStruct((_NP4 // _D, _D), f32),
    )(deg_part.reshape(2 * _NP3 // _D, _D))
    d4 = d4.reshape(_NP4)

    wcat = jnp.stack([lin1_w, conv1_w, conv2_w])
    bm = 400
    f3 = pl.pallas_call(
        _fmm_body,
        grid=(3, _N // bm),
        in_specs=[
            pl.BlockSpec((bm, _D), lambda j, i: (i, 0)),
            pl.BlockSpec((1, _D, _D), lambda j, i: (j, 0, 0)),
        ],
        out_specs=pl.BlockSpec((1, bm, _D), lambda j, i: (j, i, 0)),
        out_shape=jax.ShapeDtypeStruct((3, _N, _D), f32),
    )(x, wcat)
    f3 = f3.reshape(3 * _N, _D)

    f3t = f3.reshape(3 * _N, _NC, _DH).transpose(1, 0, 2)
    pout = pl.kernel(
        _prop_body,
        out_type=jax.ShapeDtypeStruct((_NC, _NOUT, _DH), f32),
        mesh=plsc.VectorSubcoreMesh(core_axis_name="c", subcore_axis_name="s",
                                    num_cores=_NC, num_subcores=_NS),
        compiler_params=pltpu.CompilerParams(needs_layout_passes=False,
                                             use_tc_tiling_on_sc=False),
        scratch_types=[
            pltpu.VMEM((_NP4,), f32),
            pltpu.VMEM((_PROP_CKR, _D), i32),
            pltpu.VMEM((_PROP_CKR, _D), i32),
            pltpu.VMEM((_PROP_CKR, _D), i32),
            pltpu.VMEM((_PROP_CKR, _D), i32),
            pltpu.VMEM((_PROP_CKR, _D), f32),
            pltpu.VMEM((_D, _DH), f32),
            pltpu.VMEM_SHARED((_NOUT, _DH), f32),
        ],
    )(f3t, d4, rowf, colp, grp, gcp, wp_)
    ptot = jnp.concatenate([pout[0], pout[1]], axis=1)[:_N]

    bsum = (ln_b + conv1_b + conv2_b).reshape(1, _D)
    out = pl.pallas_call(
        _final_body,
        grid=(_N // bm,),
        in_specs=[
            pl.BlockSpec((bm, _D), lambda i: (i, 0)),
            pl.BlockSpec((bm, _D), lambda i: (i, 0)),
            pl.BlockSpec((_D, _D), lambda i: (0, 0)),
            pl.BlockSpec((_D, _D), lambda i: (0, 0)),
            pl.BlockSpec((1, _D), lambda i: (0, 0)),
            pl.BlockSpec((1, _D), lambda i: (0, 0)),
            pl.BlockSpec((1, _D), lambda i: (0, 0)),
            pl.BlockSpec((1, _D), lambda i: (0, 0)),
        ],
        out_specs=pl.BlockSpec((bm, _D), lambda i: (i, 0)),
        out_shape=jax.ShapeDtypeStruct((_N, _D), f32),
    )(x, ptot, ln_w, conv_w, bsum, conv_b.reshape(1, _D),
      bn_gamma.reshape(1, _D), bn_beta.reshape(1, _D))
    return out
